# baseline (speedup 1.0000x reference)
import functools

import jax
import jax.numpy as jnp
from jax import lax
from jax.experimental import pallas as pl
from jax.experimental.pallas import tpu as pltpu
from jax.experimental.pallas import tpu_sc as plsc

NB = 8
NP = 4 * NB - 1
BOUND = 3.0
MIN_W = 0.001
MIN_H = 0.001
MIN_D = 0.001
MIN_L = 0.025
EPS = 1e-6
LN2 = 0.6931471805599453
SQRT2 = 1.4142135623730951

NC = 2
NS = 16
L = 16
NW = NC * NS

SUB = 8
LANES = 128

R_SC = 2560
TC_B = 2048


def _log_full(x):
    xi = plsc.bitcast(x, jnp.int32)
    e = (jnp.right_shift(xi, 23) & 0xFF) - 127
    m = plsc.bitcast((xi & 0x007FFFFF) | 0x3F800000, jnp.float32)
    big = m > SQRT2
    m = jnp.where(big, 0.5 * m, m)
    e = jnp.where(big, e + 1, e)
    z = (m - 1.0) / (m + 1.0)
    z2 = z * z
    p = 1.0 / 3.0 + z2 * (0.2 + z2 * (1.0 / 7.0 + z2 * (1.0 / 9.0)))
    return e.astype(jnp.float32) * LN2 + (2.0 * z) * (1.0 + z2 * p)


def _log1p_small(u):
    z = u / (u + 2.0)
    z2 = z * z
    p = 1.0 / 3.0 + z2 * (0.2 + z2 * (1.0 / 7.0 + z2 * (1.0 / 9.0)))
    return (2.0 * z) * (1.0 + z2 * p)


def _softplus_sc(x):
    return jnp.maximum(x, 0.0) + _log1p_small(jnp.exp(-jnp.abs(x)))


def _sqrt_nr(r):
    yi = 0x5F3759DF - jnp.right_shift(plsc.bitcast(r, jnp.int32), 1)
    y = plsc.bitcast(yi, jnp.float32)
    y = y * (1.5 - 0.5 * r * y * y)
    y = y * (1.5 - 0.5 * r * y * y)
    y = y * (1.5 - 0.5 * r * y * y)
    return r * y


def _softplus_tc(x):
    return jnp.maximum(x, 0.0) + jnp.log1p(jnp.exp(-jnp.abs(x)))


def _spline_math(x, p, ln, sqrt, softplus):
    def knots(v, mn):
        e = [jnp.exp(v[k]) for k in range(NB)]
        s = e[0]
        for k in range(1, NB):
            s = s + e[k]
        ci = (1.0 - mn * NB) / s
        widths = [mn + ci * e[k] for k in range(NB)]
        cum = widths[0]
        K = [jnp.full_like(x, -BOUND), 6.0 * cum - BOUND]
        for k in range(1, NB - 1):
            cum = cum + widths[k]
            K.append(6.0 * cum - BOUND)
        K.append(jnp.full_like(x, BOUND))
        return K

    Kw = knots(p[0:8], MIN_W)
    Kh = knots(p[8:16], MIN_H)
    dr = p[16:23]
    lr = p[23:31]

    b = [Kw[j] + EPS <= x for j in range(1, 8)]

    def sel8(v, lo=1, hi=8):
        r = v[lo - 1]
        for j in range(lo, hi):
            r = jnp.where(b[j - 1], v[j], r)
        return r

    Kw_s = sel8(Kw[0:8])
    W_s = sel8(Kw[1:9]) - Kw_s
    ya = sel8(Kh[0:8])
    yb = sel8(Kh[1:9])
    H_s = yb - ya
    dr_lo = sel8([dr[0]] + dr, lo=2)
    dr_hi = sel8(dr + [dr[6]], hi=7)
    D_s = jnp.where(b[0], MIN_D + softplus(dr_lo), 1.0 - MIN_D)
    Dp1_s = jnp.where(b[6], 1.0 - MIN_D, MIN_D + softplus(dr_hi))
    lam = (1.0 - 2.0 * MIN_L) / (1.0 + jnp.exp(-sel8(lr))) + MIN_L

    rW = 1.0 / W_s
    wb = sqrt(D_s / Dp1_s)
    lwb = lam * wb
    wc = (lam * D_s + (wb - lwb) * Dp1_s) * W_s / H_s
    l1 = 1.0 - lam
    yc = (lwb * yb + l1 * ya) / (l1 + lwb)
    theta = (x - Kw_s) * rW
    ind = theta <= lam
    ltheta = lam - theta
    wcyc = wc * yc
    wcyctheta = wcyc * theta
    num = jnp.where(ind, wcyctheta + ya * ltheta,
                    (wcyc - wcyctheta) - (wb * yb) * ltheta)
    wctheta = wc * theta
    den = jnp.where(ind, wctheta + ltheta, (wc - wctheta) - wb * ltheta)
    out = num / den
    dnum = wc * jnp.where(ind, lam * (yc - ya), (wb - lwb) * (yb - yc)) * rW
    lad = ln(dnum / (den * den))
    outside = (x < -BOUND) | (x > BOUND)
    out = jnp.where(outside, x, out)
    lad = jnp.where(outside, 0.0, lad)
    return out, lad


def _spline_group(x, p):
    return _spline_math(x, p, _log_full, _sqrt_nr, _softplus_sc)


@jax.jit
def _sc_spline(x2, p3):
    nc, nr = x2.shape
    n_rb = R_SC // LANES
    per_w = (nc // SUB) * n_rb // NW
    mesh = plsc.VectorSubcoreMesh(core_axis_name="c", subcore_axis_name="s")

    @functools.partial(
        pl.kernel,
        mesh=mesh,
        compiler_params=pltpu.CompilerParams(needs_layout_passes=False),
        out_type=(
            jax.ShapeDtypeStruct((nc, R_SC), jnp.float32),
            jax.ShapeDtypeStruct((nc, R_SC), jnp.float32),
        ),
        scratch_types=[
            pltpu.VMEM((NP, SUB, LANES), jnp.float32),
            pltpu.VMEM((NP, SUB, LANES), jnp.float32),
            pltpu.VMEM((SUB, LANES), jnp.float32),
            pltpu.VMEM((SUB, LANES), jnp.float32),
            pltpu.VMEM((SUB, LANES), jnp.float32),
            pltpu.VMEM((SUB, LANES), jnp.float32),
            pltpu.VMEM((SUB, LANES), jnp.float32),
            pltpu.VMEM((SUB, LANES), jnp.float32),
            pltpu.SemaphoreType.DMA,
            pltpu.SemaphoreType.DMA,
            pltpu.SemaphoreType.DMA,
            pltpu.SemaphoreType.DMA,
        ],
    )
    def k(x_hbm, p_hbm, out_hbm, lad_hbm,
          pv0, pv1, xv0, xv1, ov0, lv0, ov1, lv1,
          isem0, isem1, osem0, osem1):
        wid = lax.axis_index("s") * NC + lax.axis_index("c")
        base = wid * per_w

        def slices(i):
            ch = base + i
            c0 = (ch // n_rb) * SUB
            r0 = (ch % n_rb) * LANES
            return pl.ds(c0, SUB), pl.ds(r0, LANES)

        def fire_in(i, pv, xv, isem):
            cs, rs = slices(i)
            pltpu.async_copy(p_hbm.at[:, cs, rs], pv, isem)
            pltpu.async_copy(x_hbm.at[cs, rs], xv, isem)

        def wait_in(pv, xv, isem):
            pltpu.make_async_copy(p_hbm.at[:, pl.ds(0, SUB), pl.ds(0, LANES)],
                                  pv, isem).wait()
            pltpu.make_async_copy(x_hbm.at[pl.ds(0, SUB), pl.ds(0, LANES)],
                                  xv, isem).wait()

        def fire_out(i, ov, lv, osem):
            cs, rs = slices(i)
            pltpu.async_copy(ov, out_hbm.at[cs, rs], osem)
            pltpu.async_copy(lv, lad_hbm.at[cs, rs], osem)

        def wait_out(ov, lv, osem):
            pltpu.make_async_copy(out_hbm.at[pl.ds(0, SUB), pl.ds(0, LANES)],
                                  ov, osem).wait()
            pltpu.make_async_copy(lad_hbm.at[pl.ds(0, SUB), pl.ds(0, LANES)],
                                  lv, osem).wait()

        def compute(pv, xv, ov, lv):
            def do_group(ci, gi):
                p = [pv[kk, ci, pl.ds(gi * L, L)] for kk in range(NP)]
                xg = xv[ci, pl.ds(gi * L, L)]
                o, lad = _spline_group(xg, p)
                ov[ci, pl.ds(gi * L, L)] = o
                lv[ci, pl.ds(gi * L, L)] = lad

            def group_body(gg, carry2):
                ci = gg // (LANES // (2 * L))
                gi = gg % (LANES // (2 * L))
                do_group(ci, 2 * gi)
                do_group(ci, 2 * gi + 1)
                return carry2

            lax.fori_loop(0, SUB * LANES // (2 * L), group_body, 0)

        fire_in(0, pv0, xv0, isem0)

        def pipe_body(t, carry):
            fire_in(2 * t + 1, pv1, xv1, isem1)
            wait_in(pv0, xv0, isem0)

            @pl.when(t > 0)
            def _():
                wait_out(ov0, lv0, osem0)

            compute(pv0, xv0, ov0, lv0)
            fire_out(2 * t, ov0, lv0, osem0)

            @pl.when(2 * t + 2 < per_w)
            def _():
                fire_in(2 * t + 2, pv0, xv0, isem0)

            wait_in(pv1, xv1, isem1)

            @pl.when(t > 0)
            def _():
                wait_out(ov1, lv1, osem1)

            compute(pv1, xv1, ov1, lv1)
            fire_out(2 * t + 1, ov1, lv1, osem1)
            return carry

        lax.fori_loop(0, per_w // 2, pipe_body, 0)
        if per_w % 2 == 1:
            wait_in(pv0, xv0, isem0)
            if per_w > 1:
                wait_out(ov0, lv0, osem0)
            compute(pv0, xv0, ov0, lv0)
            fire_out(per_w - 1, ov0, lv0, osem0)
        wait_out(ov0, lv0, osem0)
        wait_out(ov1, lv1, osem1)

    return k(x2, p3)


def _tc_body(x_ref, p_ref, out_ref, lad_ref):
    x = x_ref[...]
    p = [p_ref[kk] for kk in range(NP)]
    o, lad = _spline_math(x, p, jnp.log, jnp.sqrt, _softplus_tc)
    out_ref[...] = o
    lad_ref[...] = lad


@jax.jit
def _tc_spline(x2, p3):
    nc, nr = x2.shape
    grid = ((nr - R_SC) // TC_B,)
    off = R_SC // TC_B
    return pl.pallas_call(
        _tc_body,
        grid=grid,
        in_specs=[
            pl.BlockSpec((nc, TC_B), lambda i: (0, i + off)),
            pl.BlockSpec((NP, nc, TC_B), lambda i: (0, 0, i + off)),
        ],
        out_specs=[
            pl.BlockSpec((nc, TC_B), lambda i: (0, i + off)),
            pl.BlockSpec((nc, TC_B), lambda i: (0, i + off)),
        ],
        out_shape=[
            jax.ShapeDtypeStruct((nc, nr), jnp.float32),
            jax.ShapeDtypeStruct((nc, nr), jnp.float32),
        ],
    )(x2, p3)


def kernel(inputs, params_unnorm):
    x2 = jnp.transpose(inputs)
    p3 = jnp.transpose(params_unnorm, (2, 1, 0))
    out_sc, lad_sc = _sc_spline(x2, p3)
    out_tc, lad_tc = _tc_spline(x2, p3)
    out = lax.dynamic_update_slice(out_tc, out_sc, (0, 0))
    lad = lax.dynamic_update_slice(lad_tc, lad_sc, (0, 0))
    return jnp.transpose(out), jnp.transpose(lad)


# speedup vs baseline: 1.1526x; 1.1526x over previous
import functools

import jax
import jax.numpy as jnp
from jax import lax
from jax.experimental import pallas as pl
from jax.experimental.pallas import tpu as pltpu
from jax.experimental.pallas import tpu_sc as plsc

NB = 8
NP = 4 * NB - 1
BOUND = 3.0
MIN_W = 0.001
MIN_H = 0.001
MIN_D = 0.001
MIN_L = 0.025
EPS = 1e-6
LN2 = 0.6931471805599453
SQRT2 = 1.4142135623730951

NC = 2
NS = 16
L = 16
NW = NC * NS

SUB = 8
LANES = 128

R_SC = 2048
TC_B = 2048


def _log_full(x):
    xi = plsc.bitcast(x, jnp.int32)
    e = (jnp.right_shift(xi, 23) & 0xFF) - 127
    m = plsc.bitcast((xi & 0x007FFFFF) | 0x3F800000, jnp.float32)
    big = m > SQRT2
    m = jnp.where(big, 0.5 * m, m)
    e = jnp.where(big, e + 1, e)
    z = (m - 1.0) / (m + 1.0)
    z2 = z * z
    p = 1.0 / 3.0 + z2 * (0.2 + z2 * (1.0 / 7.0 + z2 * (1.0 / 9.0)))
    return e.astype(jnp.float32) * LN2 + (2.0 * z) * (1.0 + z2 * p)


def _log1p_small(u):
    z = u / (u + 2.0)
    z2 = z * z
    p = 1.0 / 3.0 + z2 * (0.2 + z2 * (1.0 / 7.0 + z2 * (1.0 / 9.0)))
    return (2.0 * z) * (1.0 + z2 * p)


def _softplus_sc(x):
    return jnp.maximum(x, 0.0) + _log1p_small(jnp.exp(-jnp.abs(x)))


def _sqrt_nr(r):
    yi = 0x5F3759DF - jnp.right_shift(plsc.bitcast(r, jnp.int32), 1)
    y = plsc.bitcast(yi, jnp.float32)
    y = y * (1.5 - 0.5 * r * y * y)
    y = y * (1.5 - 0.5 * r * y * y)
    y = y * (1.5 - 0.5 * r * y * y)
    return r * y


def _softplus_tc(x):
    return jnp.maximum(x, 0.0) + jnp.log1p(jnp.exp(-jnp.abs(x)))


def _spline_math(x, p, ln, sqrt, softplus):
    def knots(v, mn):
        e = [jnp.exp(v[k]) for k in range(NB)]
        s = e[0]
        for k in range(1, NB):
            s = s + e[k]
        ci = (1.0 - mn * NB) / s
        widths = [mn + ci * e[k] for k in range(NB)]
        cum = widths[0]
        K = [jnp.full_like(x, -BOUND), 6.0 * cum - BOUND]
        for k in range(1, NB - 1):
            cum = cum + widths[k]
            K.append(6.0 * cum - BOUND)
        K.append(jnp.full_like(x, BOUND))
        return K

    Kw = knots(p[0:8], MIN_W)
    Kh = knots(p[8:16], MIN_H)
    dr = p[16:23]
    lr = p[23:31]

    b = [Kw[j] + EPS <= x for j in range(1, 8)]

    def sel8(v, lo=1, hi=8):
        r = v[lo - 1]
        for j in range(lo, hi):
            r = jnp.where(b[j - 1], v[j], r)
        return r

    Kw_s = sel8(Kw[0:8])
    W_s = sel8(Kw[1:9]) - Kw_s
    ya = sel8(Kh[0:8])
    yb = sel8(Kh[1:9])
    H_s = yb - ya
    dr_lo = sel8([dr[0]] + dr, lo=2)
    dr_hi = sel8(dr + [dr[6]], hi=7)
    D_s = jnp.where(b[0], MIN_D + softplus(dr_lo), 1.0 - MIN_D)
    Dp1_s = jnp.where(b[6], 1.0 - MIN_D, MIN_D + softplus(dr_hi))
    lam = (1.0 - 2.0 * MIN_L) / (1.0 + jnp.exp(-sel8(lr))) + MIN_L

    rW = 1.0 / W_s
    wb = sqrt(D_s / Dp1_s)
    lwb = lam * wb
    wc = (lam * D_s + (wb - lwb) * Dp1_s) * W_s / H_s
    l1 = 1.0 - lam
    yc = (lwb * yb + l1 * ya) / (l1 + lwb)
    theta = (x - Kw_s) * rW
    ind = theta <= lam
    ltheta = lam - theta
    wcyc = wc * yc
    wcyctheta = wcyc * theta
    num = jnp.where(ind, wcyctheta + ya * ltheta,
                    (wcyc - wcyctheta) - (wb * yb) * ltheta)
    wctheta = wc * theta
    den = jnp.where(ind, wctheta + ltheta, (wc - wctheta) - wb * ltheta)
    out = num / den
    dnum = wc * jnp.where(ind, lam * (yc - ya), (wb - lwb) * (yb - yc)) * rW
    lad = ln(dnum / (den * den))
    outside = (x < -BOUND) | (x > BOUND)
    out = jnp.where(outside, x, out)
    lad = jnp.where(outside, 0.0, lad)
    return out, lad


def _spline_group(x, p):
    return _spline_math(x, p, _log_full, _sqrt_nr, _softplus_sc)


@jax.jit
def _sc_spline(x2, p3):
    nc, nr = x2.shape
    n_rb = R_SC // LANES
    per_w = (nc // SUB) * n_rb // NW
    mesh = plsc.VectorSubcoreMesh(core_axis_name="c", subcore_axis_name="s")

    @functools.partial(
        pl.kernel,
        mesh=mesh,
        compiler_params=pltpu.CompilerParams(needs_layout_passes=False),
        out_type=(
            jax.ShapeDtypeStruct((nc, R_SC), jnp.float32),
            jax.ShapeDtypeStruct((nc, R_SC), jnp.float32),
        ),
        scratch_types=[
            pltpu.VMEM((NP, SUB, LANES), jnp.float32),
            pltpu.VMEM((NP, SUB, LANES), jnp.float32),
            pltpu.VMEM((SUB, LANES), jnp.float32),
            pltpu.VMEM((SUB, LANES), jnp.float32),
            pltpu.VMEM((SUB, LANES), jnp.float32),
            pltpu.VMEM((SUB, LANES), jnp.float32),
            pltpu.VMEM((SUB, LANES), jnp.float32),
            pltpu.VMEM((SUB, LANES), jnp.float32),
            pltpu.SemaphoreType.DMA,
            pltpu.SemaphoreType.DMA,
            pltpu.SemaphoreType.DMA,
            pltpu.SemaphoreType.DMA,
        ],
    )
    def k(x_hbm, p_hbm, out_hbm, lad_hbm,
          pv0, pv1, xv0, xv1, ov0, lv0, ov1, lv1,
          isem0, isem1, osem0, osem1):
        wid = lax.axis_index("s") * NC + lax.axis_index("c")
        base = wid * per_w

        def slices(i):
            ch = base + i
            c0 = (ch // n_rb) * SUB
            r0 = (ch % n_rb) * LANES
            return pl.ds(c0, SUB), pl.ds(r0, LANES)

        def fire_in(i, pv, xv, isem):
            cs, rs = slices(i)
            pltpu.async_copy(p_hbm.at[:, cs, rs], pv, isem)
            pltpu.async_copy(x_hbm.at[cs, rs], xv, isem)

        def wait_in(pv, xv, isem):
            pltpu.make_async_copy(p_hbm.at[:, pl.ds(0, SUB), pl.ds(0, LANES)],
                                  pv, isem).wait()
            pltpu.make_async_copy(x_hbm.at[pl.ds(0, SUB), pl.ds(0, LANES)],
                                  xv, isem).wait()

        def fire_out(i, ov, lv, osem):
            cs, rs = slices(i)
            pltpu.async_copy(ov, out_hbm.at[cs, rs], osem)
            pltpu.async_copy(lv, lad_hbm.at[cs, rs], osem)

        def wait_out(ov, lv, osem):
            pltpu.make_async_copy(out_hbm.at[pl.ds(0, SUB), pl.ds(0, LANES)],
                                  ov, osem).wait()
            pltpu.make_async_copy(lad_hbm.at[pl.ds(0, SUB), pl.ds(0, LANES)],
                                  lv, osem).wait()

        def compute(pv, xv, ov, lv):
            def do_group(ci, gi):
                p = [pv[kk, ci, pl.ds(gi * L, L)] for kk in range(NP)]
                xg = xv[ci, pl.ds(gi * L, L)]
                o, lad = _spline_group(xg, p)
                ov[ci, pl.ds(gi * L, L)] = o
                lv[ci, pl.ds(gi * L, L)] = lad

            def group_body(gg, carry2):
                ci = gg // (LANES // (2 * L))
                gi = gg % (LANES // (2 * L))
                do_group(ci, 2 * gi)
                do_group(ci, 2 * gi + 1)
                return carry2

            lax.fori_loop(0, SUB * LANES // (2 * L), group_body, 0)

        fire_in(0, pv0, xv0, isem0)

        def pipe_body(t, carry):
            fire_in(2 * t + 1, pv1, xv1, isem1)
            wait_in(pv0, xv0, isem0)

            @pl.when(t > 0)
            def _():
                wait_out(ov0, lv0, osem0)

            compute(pv0, xv0, ov0, lv0)
            fire_out(2 * t, ov0, lv0, osem0)

            @pl.when(2 * t + 2 < per_w)
            def _():
                fire_in(2 * t + 2, pv0, xv0, isem0)

            wait_in(pv1, xv1, isem1)

            @pl.when(t > 0)
            def _():
                wait_out(ov1, lv1, osem1)

            compute(pv1, xv1, ov1, lv1)
            fire_out(2 * t + 1, ov1, lv1, osem1)
            return carry

        lax.fori_loop(0, per_w // 2, pipe_body, 0)
        if per_w % 2 == 1:
            wait_in(pv0, xv0, isem0)
            if per_w > 1:
                wait_out(ov0, lv0, osem0)
            compute(pv0, xv0, ov0, lv0)
            fire_out(per_w - 1, ov0, lv0, osem0)
        wait_out(ov0, lv0, osem0)
        wait_out(ov1, lv1, osem1)

    return k(x2, p3)


def _tc_body(x_ref, p_ref, out_ref, lad_ref):
    x = x_ref[...]
    p = [p_ref[kk] for kk in range(NP)]
    o, lad = _spline_math(x, p, jnp.log, jnp.sqrt, _softplus_tc)
    out_ref[...] = o
    lad_ref[...] = lad


@jax.jit
def _tc_spline(x2, p3):
    nc, nr = x2.shape
    grid = ((nr - R_SC) // TC_B,)
    off = R_SC // TC_B
    return pl.pallas_call(
        _tc_body,
        grid=grid,
        in_specs=[
            pl.BlockSpec((nc, TC_B), lambda i: (0, i + off)),
            pl.BlockSpec((NP, nc, TC_B), lambda i: (0, 0, i + off)),
        ],
        out_specs=[
            pl.BlockSpec((nc, TC_B), lambda i: (0, i + off)),
            pl.BlockSpec((nc, TC_B), lambda i: (0, i + off)),
        ],
        out_shape=[
            jax.ShapeDtypeStruct((nc, nr), jnp.float32),
            jax.ShapeDtypeStruct((nc, nr), jnp.float32),
        ],
    )(x2, p3)


def kernel(inputs, params_unnorm):
    x2 = jnp.transpose(inputs)
    p3 = jnp.transpose(params_unnorm, (2, 1, 0))
    out_sc, lad_sc = _sc_spline(x2, p3)
    out_tc, lad_tc = _tc_spline(x2, p3)
    out = lax.dynamic_update_slice(out_tc, out_sc, (0, 0))
    lad = lax.dynamic_update_slice(lad_tc, lad_sc, (0, 0))
    return jnp.transpose(out), jnp.transpose(lad)
